# baseline (speedup 1.0000x reference)
import jax
import jax.numpy as jnp
from jax.experimental import pallas as pl
from jax.experimental.pallas import tpu as pltpu

BLOCK_N = 2048


def _logits_kernel(f_ref, m_ref, o_ref):
    o_ref[...] = jax.lax.dot_general(
        f_ref[...].astype(jnp.bfloat16),
        m_ref[...].astype(jnp.bfloat16),
        dimension_numbers=(((1,), (1,)), ((), ())),
        preferred_element_type=jnp.float32,
    )


def kernel(feature, index, memory, temperature, momentum):
    b, k = feature.shape
    n = memory.shape[0]
    f_scaled = feature * (1.0 / temperature)
    grid = pl.cdiv(n, BLOCK_N)
    return pl.pallas_call(
        _logits_kernel,
        grid=(grid,),
        in_specs=[
            pl.BlockSpec((b, k), lambda i: (0, 0)),
            pl.BlockSpec((BLOCK_N, k), lambda i: (i, 0)),
        ],
        out_specs=pl.BlockSpec((b, BLOCK_N), lambda i: (0, i)),
        out_shape=jax.ShapeDtypeStruct((b, n), jnp.float32),
        compiler_params=pltpu.CompilerParams(
            dimension_semantics=("parallel",),
        ),
    )(f_scaled, memory)


# speedup vs baseline: 1.1315x; 1.1315x over previous
import jax
import jax.numpy as jnp
from jax.experimental import pallas as pl
from jax.experimental.pallas import tpu as pltpu

BLOCK_M = 32


def _logits_kernel(f_ref, mt_ref, o_ref):
    o_ref[...] = jax.lax.dot_general(
        f_ref[...].astype(jnp.bfloat16),
        mt_ref[...].astype(jnp.bfloat16),
        dimension_numbers=(((1,), (0,)), ((), ())),
        preferred_element_type=jnp.float32,
    )


def kernel(feature, index, memory, temperature, momentum):
    b, k = feature.shape
    n = memory.shape[0]
    f_scaled = feature * (1.0 / temperature)
    mt = memory.T
    grid = pl.cdiv(b, BLOCK_M)
    return pl.pallas_call(
        _logits_kernel,
        grid=(grid,),
        in_specs=[
            pl.BlockSpec((BLOCK_M, k), lambda i: (i, 0)),
            pl.BlockSpec((k, n), lambda i: (0, 0)),
        ],
        out_specs=pl.BlockSpec((BLOCK_M, n), lambda i: (i, 0)),
        out_shape=jax.ShapeDtypeStruct((b, n), jnp.float32),
        compiler_params=pltpu.CompilerParams(
            dimension_semantics=("parallel",),
        ),
    )(f_scaled, mt)
